# baseline (speedup 1.0000x reference)
import functools

import jax
import jax.numpy as jnp
from jax import lax
from jax.experimental import pallas as pl
from jax.experimental.pallas import tpu as pltpu
from jax.experimental.pallas import tpu_sc as plsc

VOCAB = 1000
DIM = 32
NC = 2
NS = 16
NW = NC * NS
HC = 5


def _build(batch: int, hist: int):
    assert batch % (NW * 128) == 0 and batch // 128 == NW
    assert hist % (2 * HC) == 0 and hist % 8 == 0
    n_chunks = hist // HC
    n_pairs = n_chunks // 2
    hist_hi = hist // 8

    mesh = plsc.VectorSubcoreMesh(core_axis_name="c", subcore_axis_name="s")

    @functools.partial(
        pl.kernel,
        mesh=mesh,
        compiler_params=pltpu.CompilerParams(
            use_tc_tiling_on_sc=False, needs_layout_passes=False),
        out_type=jax.ShapeDtypeStruct((hist, DIM // 8, NW, 1024), jnp.float32),
        scratch_types=[
            pltpu.VMEM((VOCAB * DIM,), jnp.float32),
            pltpu.VMEM((hist_hi, 8, 128), jnp.int32),
            pltpu.VMEM((HC, DIM // 8, 1024), jnp.float32),
            pltpu.VMEM((HC, DIM // 8, 1024), jnp.float32),
            pltpu.SemaphoreType.DMA,
            pltpu.SemaphoreType.DMA,
        ],
    )
    def emb(idx_hbm, table_hbm, out_hbm, table_v, idx_v, buf0, buf1,
            wb0, wb1):
        w = lax.axis_index("s") * NC + lax.axis_index("c")
        pltpu.sync_copy(table_hbm, table_v)
        pltpu.sync_copy(idx_hbm.at[:, w, :, :], idx_v)

        def compute_chunk(c, buf):
            @plsc.parallel_loop(0, 8)
            def g_body(g):
                g16 = g * 16
                for hh in range(HC):
                    h = c * HC + hh
                    ids = idx_v[h // 8, h % 8, pl.ds(g16, 16)]
                    base = ids * DIM
                    for e in range(DIM):
                        v = plsc.load_gather(table_v, [base + e])
                        buf[hh, e // 8, pl.ds((e % 8) * 128 + g16, 16)] = v

        def start_wb(c, buf, sem):
            pltpu.async_copy(
                buf, out_hbm.at[pl.ds(c * HC, HC), :, w, :], sem)

        def wait_wb(buf, sem):
            pltpu.make_async_copy(
                buf, out_hbm.at[pl.ds(0, HC), :, 0, :], sem).wait()

        def body(p, carry):
            e = 2 * p
            compute_chunk(e, buf0)
            compute_chunk(e + 1, buf1)
            return carry

        lax.fori_loop(0, n_pairs, body, 0)
        start_wb(0, buf0, wb0)
        wait_wb(buf0, wb0)

    return emb


def kernel(tensor, table):
    batch, hist = tensor.shape
    idx4 = tensor.reshape(NW, 128, hist // 8, 8).transpose(2, 0, 3, 1)
    raw = _build(batch, hist)(idx4, table.reshape(-1))
    x = raw.reshape(hist, DIM // 8, NW, 8, 128).transpose(2, 4, 0, 1, 3)
    return x.reshape(batch, hist, DIM)


# speedup vs baseline: 1.9792x; 1.9792x over previous
import functools

import jax
import jax.numpy as jnp
from jax import lax
from jax.experimental import pallas as pl
from jax.experimental.pallas import tpu as pltpu
from jax.experimental.pallas import tpu_sc as plsc

VOCAB = 1000
DIM = 32
NC = 2
NS = 16
NW = NC * NS
HC = 5


def _build(batch: int, hist: int):
    assert batch % (NW * 128) == 0 and batch // 128 == NW
    assert hist % (2 * HC) == 0 and hist % 8 == 0
    n_chunks = hist // HC
    n_pairs = n_chunks // 2
    hist_hi = hist // 8

    mesh = plsc.VectorSubcoreMesh(core_axis_name="c", subcore_axis_name="s")

    @functools.partial(
        pl.kernel,
        mesh=mesh,
        compiler_params=pltpu.CompilerParams(
            use_tc_tiling_on_sc=False, needs_layout_passes=False),
        out_type=jax.ShapeDtypeStruct((hist, DIM // 8, NW, 1024), jnp.float32),
        scratch_types=[
            pltpu.VMEM((VOCAB, 40), jnp.float32),
            pltpu.VMEM((hist_hi, 8, 128), jnp.int32),
            pltpu.VMEM((HC, DIM // 8, 1024), jnp.float32),
            pltpu.VMEM((HC, DIM // 8, 1024), jnp.float32),
            pltpu.SemaphoreType.DMA,
            pltpu.SemaphoreType.DMA,
        ],
    )
    def emb(idx_hbm, table_hbm, out_hbm, table_v, idx_v, buf0, buf1,
            wb0, wb1):
        w = lax.axis_index("s") * NC + lax.axis_index("c")
        pltpu.sync_copy(table_hbm, table_v.at[:, pl.ds(0, DIM)])
        pltpu.sync_copy(idx_hbm.at[:, w, :, :], idx_v)

        def compute_chunk(c, buf):
            @plsc.parallel_loop(0, 8)
            def g_body(g):
                g16 = g * 16
                for hh in range(HC):
                    h = c * HC + hh
                    ids = idx_v[h // 8, h % 8, pl.ds(g16, 16)]
                    for e in range(DIM):
                        es = jnp.full((16,), e, jnp.int32)
                        v = plsc.load_gather(table_v, [ids, es])
                        buf[hh, e // 8, pl.ds((e % 8) * 128 + g16, 16)] = v

        def start_wb(c, buf, sem):
            pltpu.async_copy(
                buf, out_hbm.at[pl.ds(c * HC, HC), :, w, :], sem)

        def wait_wb(buf, sem):
            pltpu.make_async_copy(
                buf, out_hbm.at[pl.ds(0, HC), :, 0, :], sem).wait()

        def body(p, carry):
            e = 2 * p

            @pl.when(p > 0)
            def _():
                wait_wb(buf0, wb0)

            compute_chunk(e, buf0)

            @pl.when(p > 0)
            def _():
                wait_wb(buf1, wb1)

            start_wb(e, buf0, wb0)
            compute_chunk(e + 1, buf1)
            start_wb(e + 1, buf1, wb1)
            return carry

        lax.fori_loop(0, n_pairs, body, 0)
        wait_wb(buf0, wb0)
        wait_wb(buf1, wb1)

    return emb


def kernel(tensor, table):
    batch, hist = tensor.shape
    idx4 = tensor.reshape(NW, 128, hist // 8, 8).transpose(2, 0, 3, 1)
    raw = _build(batch, hist)(idx4, table)
    x = raw.reshape(hist, DIM // 8, NW, 8, 128).transpose(2, 4, 0, 1, 3)
    return x.reshape(batch, hist, DIM)


# speedup vs baseline: 2.9691x; 1.5002x over previous
import functools

import jax
import jax.numpy as jnp
from jax import lax
from jax.experimental import pallas as pl
from jax.experimental.pallas import tpu as pltpu
from jax.experimental.pallas import tpu_sc as plsc

VOCAB = 1000
DIM = 32
NC = 2
NS = 16
NW = NC * NS
HC = 5


def _build(batch: int, hist: int):
    assert batch % (NW * 128) == 0 and batch // 128 == NW
    assert hist % (2 * HC) == 0 and hist % 8 == 0
    n_chunks = hist // HC
    n_pairs = n_chunks // 2
    hist_hi = hist // 8

    mesh = plsc.VectorSubcoreMesh(core_axis_name="c", subcore_axis_name="s")

    @functools.partial(
        pl.kernel,
        mesh=mesh,
        compiler_params=pltpu.CompilerParams(
            use_tc_tiling_on_sc=False, needs_layout_passes=False),
        out_type=jax.ShapeDtypeStruct((hist, DIM // 8, NW, 1024), jnp.float32),
        scratch_types=[
            pltpu.VMEM((VOCAB, 40), jnp.float32),
            pltpu.VMEM((hist_hi, 8, 128), jnp.int32),
            pltpu.VMEM((HC, DIM // 8, 1024), jnp.float32),
            pltpu.VMEM((HC, DIM // 8, 1024), jnp.float32),
            pltpu.SemaphoreType.DMA,
            pltpu.SemaphoreType.DMA,
        ],
    )
    def emb(idx_hbm, table_hbm, out_hbm, table_v, idx_v, buf0, buf1,
            wb0, wb1):
        w = lax.axis_index("s") * NC + lax.axis_index("c")
        pltpu.sync_copy(table_hbm, table_v.at[:, pl.ds(0, DIM)])
        pltpu.sync_copy(idx_hbm.at[:, w, :, :], idx_v)

        def compute_chunk(c, buf):
            @plsc.parallel_loop(0, HC * 8)
            def g_body(gi):
                hh = gi // 8
                g16 = (gi % 8) * 16
                h = c * HC + hh
                ids = idx_v[h // 8, h % 8, pl.ds(g16, 16)]
                for e in range(DIM):
                    es = jnp.full((16,), e, jnp.int32)
                    v = plsc.load_gather(table_v, [ids, es])
                    buf[hh, e // 8, pl.ds((e % 8) * 128 + g16, 16)] = v

        def start_wb(c, buf, sem):
            pltpu.async_copy(
                buf, out_hbm.at[pl.ds(c * HC, HC), :, w, :], sem)

        def wait_wb(buf, sem):
            pltpu.make_async_copy(
                buf, out_hbm.at[pl.ds(0, HC), :, 0, :], sem).wait()

        def body(p, carry):
            e = 2 * p

            @pl.when(p > 0)
            def _():
                wait_wb(buf0, wb0)

            compute_chunk(e, buf0)

            @pl.when(p > 0)
            def _():
                wait_wb(buf1, wb1)

            start_wb(e, buf0, wb0)
            compute_chunk(e + 1, buf1)
            start_wb(e + 1, buf1, wb1)
            return carry

        lax.fori_loop(0, n_pairs, body, 0)
        wait_wb(buf0, wb0)
        wait_wb(buf1, wb1)

    return emb


def kernel(tensor, table):
    batch, hist = tensor.shape
    idx4 = tensor.reshape(NW, 128, hist // 8, 8).transpose(2, 0, 3, 1)
    raw = _build(batch, hist)(idx4, table)
    x = raw.reshape(hist, DIM // 8, NW, 8, 128).transpose(2, 4, 0, 1, 3)
    return x.reshape(batch, hist, DIM)
